# baseline (speedup 1.0000x reference)
import jax
import jax.numpy as jnp
from jax import lax
from jax.experimental import pallas as pl
from jax.experimental.pallas import tpu as pltpu
from jax.experimental.pallas import tpu_sc as plsc

N = 10000
E = 320000
HID = 128
HH = HID // 2
RELS = 1000
RP = 1024
NC = 2
NS = 16
NW = NC * NS
EPW = E // NW
EPT = E // NS
BCH = 2000
CSUB = 40
NSUB = BCH // CSUB
S2R = RP * 4 // 128

_f32 = jnp.float32
_i32 = jnp.int32


def _proj_body(x_ref, wh_ref, wt_ref, ah_ref, at_ref,
               xh0_ref, xt0_ref, sh_ref, st_ref, m_ref):
    x = x_ref[...]
    xh = jnp.dot(x, wh_ref[...], preferred_element_type=_f32)
    xt = jnp.dot(x, wt_ref[...], preferred_element_type=_f32)
    xh0_ref[...] = xh
    xt0_ref[...] = xt
    ah2 = ah_ref[0:2, :]
    at2 = at_ref[0:2, :]
    dn = (((1,), (1,)), ((), ()))
    sh = jnp.concatenate(
        [lax.dot_general(xh, ah2, dn, preferred_element_type=_f32),
         lax.dot_general(xt, ah2, dn, preferred_element_type=_f32)], axis=1)
    st = jnp.concatenate(
        [lax.dot_general(xt, at2, dn, preferred_element_type=_f32),
         lax.dot_general(xh, at2, dn, preferred_element_type=_f32)], axis=1)
    sh_ref[...] = sh
    st_ref[...] = st
    m = jnp.max(jnp.max(sh, axis=0) + jnp.max(st, axis=0))
    m_ref[...] = jnp.full((1, 16), m, _f32)


def _project(x_e, w_h, w_t, a_h, a_t):
    return pl.pallas_call(
        _proj_body,
        out_shape=[
            jax.ShapeDtypeStruct((N, HID), _f32),
            jax.ShapeDtypeStruct((N, HID), _f32),
            jax.ShapeDtypeStruct((N, 4), _f32),
            jax.ShapeDtypeStruct((N, 4), _f32),
            jax.ShapeDtypeStruct((1, 16), _f32),
        ],
    )(x_e, w_h, w_t, a_h, a_t)


def _scores_body(sh_hbm, st_hbm, m_hbm, h_hbm, t_hbm, r_hbm,
                 ex_hbm, psum_hbm,
                 sh_v, st_v, m_v, h_v, t_v, r_v,
                 ex0_v, ex1_v, ex2_v, ex3_v, ssum_v, zb_v, ridx_v, sacc_sh,
                 sstg, swb):
    cid = lax.axis_index("c")
    sid = lax.axis_index("s")
    wid = sid * NC + cid
    pltpu.async_copy(sh_hbm, sh_v, sstg)
    pltpu.async_copy(st_hbm, st_v, sstg)
    pltpu.async_copy(m_hbm, m_v, sstg)
    pltpu.make_async_copy(sh_hbm, sh_v, sstg).wait()
    pltpu.make_async_copy(st_hbm, st_v, sstg).wait()
    pltpu.make_async_copy(m_hbm, m_v, sstg).wait()
    mvec = m_v[...]

    def _zero(r, _):
        for q in range(8):
            ssum_v[r, pl.ds(q * 16, 16)] = jnp.zeros((16,), _f32)
        return 0
    lax.fori_loop(0, S2R, _zero, 0)
    for k in range(2):
        for q in range(8):
            zb_v[k, pl.ds(q * 16, 16)] = jnp.zeros((16,), _f32)
    pltpu.sync_copy(zb_v, sacc_sh.at[pl.ds(sid * 2, 2)])
    iota = lax.iota(_i32, 16)
    ridx_v[0, pl.ds(0, 16)] = iota
    ridx_v[0, pl.ds(16, 16)] = iota + 16
    plsc.subcore_barrier()

    ex_refs = (ex0_v, ex1_v, ex2_v, ex3_v)
    base0 = wid * EPW
    for ch in range(5):
        base = base0 + ch * BCH
        pltpu.async_copy(h_hbm.at[pl.ds(base, BCH)], h_v, sstg)
        pltpu.async_copy(t_hbm.at[pl.ds(base, BCH)], t_v, sstg)
        pltpu.async_copy(r_hbm.at[pl.ds(base, BCH)], r_v, sstg)
        pltpu.make_async_copy(h_hbm.at[pl.ds(base, BCH)], h_v, sstg).wait()
        pltpu.make_async_copy(t_hbm.at[pl.ds(base, BCH)], t_v, sstg).wait()
        pltpu.make_async_copy(r_hbm.at[pl.ds(base, BCH)], r_v, sstg).wait()
        if ch > 0:
            pbase = base0 + (ch - 1) * BCH
            for j in range(4):
                pltpu.make_async_copy(
                    ex_refs[j], ex_hbm.at[pl.ds(j * E + pbase, BCH)],
                    swb).wait()

        def _edges(i, _):
            sl = pl.ds(i * 16, 16)
            h4 = h_v[sl] * 4
            t4 = t_v[sl] * 4
            rv = r_v[sl]
            rrow = rv >> 5
            rcol = (rv & 31) * 4
            for j in range(4):
                e = (plsc.load_gather(sh_v, [h4 + j])
                     + plsc.load_gather(st_v, [t4 + j]))
                e = jnp.where(e >= 0.0, e, e * 0.01)
                ex = jnp.exp(e - mvec)
                ex_refs[j][sl] = ex
                plsc.addupdate_scatter(ssum_v, [rrow, rcol + j], ex)
            return 0
        lax.fori_loop(0, BCH // 16, _edges, 0)
        for j in range(4):
            pltpu.async_copy(ex_refs[j], ex_hbm.at[pl.ds(j * E + base, BCH)],
                             swb)
    for j in range(4):
        pltpu.make_async_copy(
            ex_refs[j], ex_hbm.at[pl.ds(j * E + base0 + 4 * BCH, BCH)],
            swb).wait()

    pltpu.sync_copy(ssum_v, sacc_sh.at[ridx_v.at[0]], add=True)
    plsc.subcore_barrier()

    @pl.when(sid == 0)
    def _dump():
        pltpu.sync_copy(sacc_sh, ssum_v)
        pltpu.sync_copy(ssum_v, psum_hbm.at[cid])


def _scores(sh_flat, st_flat, m_vec, h_idx, t_idx, rel):
    mesh = plsc.VectorSubcoreMesh(core_axis_name="c", subcore_axis_name="s",
                                  num_cores=NC, num_subcores=NS)
    fn = pl.kernel(
        _scores_body,
        out_type=[
            jax.ShapeDtypeStruct((4 * E,), _f32),
            jax.ShapeDtypeStruct((NC, S2R, 128), _f32),
        ],
        mesh=mesh,
        compiler_params=pltpu.CompilerParams(needs_layout_passes=False),
        scratch_types=[
            pltpu.VMEM((4 * N,), _f32),
            pltpu.VMEM((4 * N,), _f32),
            pltpu.VMEM((16,), _f32),
            pltpu.VMEM((BCH,), _i32),
            pltpu.VMEM((BCH,), _i32),
            pltpu.VMEM((BCH,), _i32),
            pltpu.VMEM((BCH,), _f32),
            pltpu.VMEM((BCH,), _f32),
            pltpu.VMEM((BCH,), _f32),
            pltpu.VMEM((BCH,), _f32),
            pltpu.VMEM((S2R, 128), _f32),
            pltpu.VMEM((2, 128), _f32),
            pltpu.VMEM((1, 32), _i32),
            pltpu.VMEM_SHARED((S2R, 128), _f32),
            pltpu.SemaphoreType.DMA,
            pltpu.SemaphoreType.DMA,
        ],
    )
    return fn(sh_flat, st_flat, m_vec, h_idx, t_idx, rel)


def _agg_body(xhs_hbm, xts_hbm, ex_hbm, h_hbm, t_hbm, r_hbm, psum_hbm,
              outp_hbm,
              inv_v, h_v, t_v, r_v,
              ex0_v, ex1_v, ex2_v, ex3_v,
              bh0_v, bh1_v, bt0_v, bt1_v, acc_v, ridx_v, acc_sh,
              sgh0, sgh1, sgt0, sgt1, sstg):
    cid = lax.axis_index("c")
    sid = lax.axis_index("s")
    iota = lax.iota(_i32, 16)
    coff = cid * HH

    pltpu.sync_copy(psum_hbm.at[0], bh0_v.at[pl.ds(0, S2R)])

    def _s0(r, _):
        for q in range(8):
            sl = pl.ds(q * 16, 16)
            inv_v[pl.ds(r * 128 + q * 16, 16)] = bh0_v[r, sl]
        return 0
    lax.fori_loop(0, S2R, _s0, 0)
    pltpu.sync_copy(psum_hbm.at[1], bh0_v.at[pl.ds(0, S2R)])

    def _inv(r, _):
        for q in range(8):
            sl = pl.ds(r * 128 + q * 16, 16)
            inv_v[sl] = 1.0 / (inv_v[sl] + bh0_v[r, pl.ds(q * 16, 16)]
                               + 1e-16)
        return 0
    lax.fori_loop(0, S2R, _inv, 0)

    def _zacc(r, _):
        for q in range(8):
            acc_v[r, pl.ds(q * 16, 16)] = jnp.zeros((16,), _f32)
        return 0
    lax.fori_loop(0, RP // 2, _zacc, 0)
    pltpu.sync_copy(acc_v.at[pl.ds(0, 32)], acc_sh.at[pl.ds(sid * 32, 32)])
    for k in range(4):
        for q in range(8):
            ridx_v[k, pl.ds(q * 16, 16)] = k * 128 + q * 16 + iota
    plsc.subcore_barrier()

    bh = (bh0_v, bh1_v)
    bt = (bt0_v, bt1_v)
    sgh = (sgh0, sgh1)
    sgt = (sgt0, sgt1)
    ex_refs = (ex0_v, ex1_v, ex2_v, ex3_v)
    base0 = sid * EPT

    def _gissue(slot, sub):
        gb = pl.ds(sub * CSUB, CSUB)
        pltpu.async_copy(xhs_hbm.at[h_v.at[gb]], bh[slot], sgh[slot])
        pltpu.async_copy(xts_hbm.at[t_v.at[gb]], bt[slot], sgt[slot])

    def _gwait(slot, sub):
        gb = pl.ds(sub * CSUB, CSUB)
        pltpu.make_async_copy(xhs_hbm.at[h_v.at[gb]], bh[slot], sgh[slot]).wait()
        pltpu.make_async_copy(xts_hbm.at[t_v.at[gb]], bt[slot], sgt[slot]).wait()

    iotav = [iota + v * 16 for v in range(4)]

    def _chunk(ch, _):
        base = base0 + ch * BCH
        pltpu.async_copy(h_hbm.at[pl.ds(base, BCH)], h_v, sstg)
        pltpu.async_copy(t_hbm.at[pl.ds(base, BCH)], t_v, sstg)
        pltpu.async_copy(r_hbm.at[pl.ds(base, BCH)], r_v, sstg)
        for j in range(4):
            pltpu.async_copy(ex_hbm.at[pl.ds(j * E + base, BCH)],
                             ex_refs[j], sstg)
        pltpu.make_async_copy(h_hbm.at[pl.ds(base, BCH)], h_v, sstg).wait()
        pltpu.make_async_copy(t_hbm.at[pl.ds(base, BCH)], t_v, sstg).wait()
        pltpu.make_async_copy(r_hbm.at[pl.ds(base, BCH)], r_v, sstg).wait()
        for j in range(4):
            pltpu.make_async_copy(ex_hbm.at[pl.ds(j * E + base, BCH)],
                                  ex_refs[j], sstg).wait()

        def _wcomb(i, _):
            sl = pl.ds(i * 16, 16)
            r4 = r_v[sl] * 4
            i0 = plsc.load_gather(inv_v, [r4])
            i1 = plsc.load_gather(inv_v, [r4 + 1])
            i2 = plsc.load_gather(inv_v, [r4 + 2])
            i3 = plsc.load_gather(inv_v, [r4 + 3])
            ex0_v[sl] = ex0_v[sl] * i0 + ex1_v[sl] * i1
            ex2_v[sl] = ex2_v[sl] * i2 + ex3_v[sl] * i3
            return 0
        lax.fori_loop(0, BCH // 16, _wcomb, 0)

        _gissue(0, 0)
        _gissue(1, 1)

        def _do_sub(slot, sub):
            _gwait(slot, sub)

            def _edge(e5, _):
                for u in range(5):
                    e = e5 * 5 + u
                    eb = jnp.full((16,), sub * CSUB + e, _i32)
                    relb = plsc.load_gather(r_v, [eb])
                    whb = plsc.load_gather(ex0_v, [eb])
                    wtb = plsc.load_gather(ex2_v, [eb])
                    arow = relb >> 1
                    acolb = (relb & 1) * HH
                    for v in range(4):
                        slv = pl.ds(coff + v * 16, 16)
                        acol = acolb + iotav[v]
                        ph = bh[slot][e, slv] * whb
                        pt = bt[slot][e, slv] * wtb
                        plsc.addupdate_scatter(acc_v, [arow, acol], ph)
                        plsc.addupdate_scatter(acc_v, [arow, acol], pt)
                return 0
            lax.fori_loop(0, CSUB // 5, _edge, 0)

        def _pair(p, _):
            for b in range(2):
                sub = p * 2 + b
                _do_sub(b, sub)
                _gissue(b, sub + 2)
            return 0
        if NSUB % 2 == 0:
            lax.fori_loop(0, NSUB // 2 - 1, _pair, 0)
            _do_sub(0, NSUB - 2)
            _do_sub(1, NSUB - 1)
        else:
            lax.fori_loop(0, (NSUB - 3) // 2, _pair, 0)
            _do_sub(0, NSUB - 3)
            _gissue(0, NSUB - 1)
            _do_sub(1, NSUB - 2)
            _do_sub(0, NSUB - 1)
        return 0
    lax.fori_loop(0, EPT // BCH, _chunk, 0)

    plsc.subcore_barrier()
    for k in range(4):
        pltpu.sync_copy(acc_v.at[pl.ds(k * 128, 128)],
                        acc_sh.at[ridx_v.at[k]], add=True)
    plsc.subcore_barrier()
    pltpu.sync_copy(acc_sh.at[pl.ds(sid * 32, 32)], acc_v.at[pl.ds(0, 32)])
    pltpu.sync_copy(acc_v.at[pl.ds(0, 32)], outp_hbm.at[cid, pl.ds(sid * 32, 32)])


def _aggregate(xhs, xts, ex_flat, h_idx, t_idx, rel, psum):
    mesh = plsc.VectorSubcoreMesh(core_axis_name="c", subcore_axis_name="s",
                                  num_cores=NC, num_subcores=NS)
    fn = pl.kernel(
        _agg_body,
        out_type=jax.ShapeDtypeStruct((NC, RP // 2, HID), _f32),
        mesh=mesh,
        compiler_params=pltpu.CompilerParams(needs_layout_passes=False),
        scratch_types=[
            pltpu.VMEM((S2R * 128,), _f32),
            pltpu.VMEM((BCH,), _i32),
            pltpu.VMEM((BCH,), _i32),
            pltpu.VMEM((BCH,), _i32),
            pltpu.VMEM((BCH,), _f32),
            pltpu.VMEM((BCH,), _f32),
            pltpu.VMEM((BCH,), _f32),
            pltpu.VMEM((BCH,), _f32),
            pltpu.VMEM((CSUB, HID), _f32),
            pltpu.VMEM((CSUB, HID), _f32),
            pltpu.VMEM((CSUB, HID), _f32),
            pltpu.VMEM((CSUB, HID), _f32),
            pltpu.VMEM((RP // 2, HID), _f32),
            pltpu.VMEM((4, 128), _i32),
            pltpu.VMEM_SHARED((RP // 2, HID), _f32),
            pltpu.SemaphoreType.DMA,
            pltpu.SemaphoreType.DMA,
            pltpu.SemaphoreType.DMA,
            pltpu.SemaphoreType.DMA,
            pltpu.SemaphoreType.DMA,
        ],
    )
    return fn(xhs, xts, ex_flat, h_idx, t_idx, rel, psum)


def _final_body(p_ref, s_ref, o_ref):
    s = s_ref[0, 0]
    o_ref[...] = jnp.concatenate(
        [p_ref[0, :RELS, :], p_ref[1, :RELS, :]], axis=1) * s


def _finalize(outp, scale):
    return pl.pallas_call(
        _final_body,
        out_shape=jax.ShapeDtypeStruct((RELS, HID), _f32),
    )(outp, scale)


def kernel(x_e, edge_index, rel, num_heads, w_h, w_t, a_h, a_t):
    h_idx = edge_index[0]
    t_idx = edge_index[1]
    rel = rel.astype(_i32)
    xhs, xts, sh, st, m = _project(x_e, w_h, w_t, a_h, a_t)
    ex_flat, psum = _scores(sh.reshape(-1), st.reshape(-1), m.reshape(16),
                            h_idx, t_idx, rel)
    outp = _aggregate(xhs, xts, ex_flat, h_idx, t_idx, rel, psum)
    outp = outp.reshape(NC, RP, HH)
    scale = (0.5 / jnp.asarray(num_heads, _f32)).reshape(1, 1)
    return _finalize(outp, scale)


# speedup vs baseline: 1.0150x; 1.0150x over previous
import jax
import jax.numpy as jnp
from jax import lax
from jax.experimental import pallas as pl
from jax.experimental.pallas import tpu as pltpu
from jax.experimental.pallas import tpu_sc as plsc

N = 10000
E = 320000
HID = 128
HH = HID // 2
RELS = 1000
RP = 1024
NC = 2
NS = 16
NW = NC * NS
EPW = E // NW
EPT = E // NS
BCH = 2000
CSUB = 40
NSUB = BCH // CSUB
S2R = RP * 4 // 128

_f32 = jnp.float32
_i32 = jnp.int32


def _proj_body(x_ref, wh_ref, wt_ref, ah_ref, at_ref,
               xh0_ref, xt0_ref, sh_ref, st_ref, m_ref):
    x = x_ref[...]
    xh = jnp.dot(x, wh_ref[...], preferred_element_type=_f32)
    xt = jnp.dot(x, wt_ref[...], preferred_element_type=_f32)
    xh0_ref[...] = xh
    xt0_ref[...] = xt
    ah2 = ah_ref[0:2, :]
    at2 = at_ref[0:2, :]
    dn = (((1,), (1,)), ((), ()))
    sh = jnp.concatenate(
        [lax.dot_general(xh, ah2, dn, preferred_element_type=_f32),
         lax.dot_general(xt, ah2, dn, preferred_element_type=_f32)], axis=1)
    st = jnp.concatenate(
        [lax.dot_general(xt, at2, dn, preferred_element_type=_f32),
         lax.dot_general(xh, at2, dn, preferred_element_type=_f32)], axis=1)
    sh_ref[...] = sh
    st_ref[...] = st
    m = jnp.max(jnp.max(sh, axis=0) + jnp.max(st, axis=0))
    m_ref[...] = jnp.full((1, 16), m, _f32)


def _project(x_e, w_h, w_t, a_h, a_t):
    return pl.pallas_call(
        _proj_body,
        out_shape=[
            jax.ShapeDtypeStruct((N, HID), _f32),
            jax.ShapeDtypeStruct((N, HID), _f32),
            jax.ShapeDtypeStruct((N, 4), _f32),
            jax.ShapeDtypeStruct((N, 4), _f32),
            jax.ShapeDtypeStruct((1, 16), _f32),
        ],
    )(x_e, w_h, w_t, a_h, a_t)


def _scores_body(sh_hbm, st_hbm, m_hbm, h_hbm, t_hbm, r_hbm,
                 ex_hbm, psum_hbm,
                 sh_v, st_v, m_v, h_v, t_v, r_v,
                 ex0_v, ex1_v, ex2_v, ex3_v, ssum_v, zb_v, ridx_v, sacc_sh,
                 sstg, swb):
    cid = lax.axis_index("c")
    sid = lax.axis_index("s")
    wid = sid * NC + cid
    pltpu.async_copy(sh_hbm, sh_v, sstg)
    pltpu.async_copy(st_hbm, st_v, sstg)
    pltpu.async_copy(m_hbm, m_v, sstg)
    pltpu.make_async_copy(sh_hbm, sh_v, sstg).wait()
    pltpu.make_async_copy(st_hbm, st_v, sstg).wait()
    pltpu.make_async_copy(m_hbm, m_v, sstg).wait()
    mvec = m_v[...]

    def _zero(r, _):
        for q in range(8):
            ssum_v[r, pl.ds(q * 16, 16)] = jnp.zeros((16,), _f32)
        return 0
    lax.fori_loop(0, S2R, _zero, 0)
    for k in range(2):
        for q in range(8):
            zb_v[k, pl.ds(q * 16, 16)] = jnp.zeros((16,), _f32)
    pltpu.sync_copy(zb_v, sacc_sh.at[pl.ds(sid * 2, 2)])
    iota = lax.iota(_i32, 16)
    ridx_v[0, pl.ds(0, 16)] = iota
    ridx_v[0, pl.ds(16, 16)] = iota + 16
    plsc.subcore_barrier()

    ex_refs = (ex0_v, ex1_v, ex2_v, ex3_v)
    base0 = wid * EPW
    for ch in range(5):
        base = base0 + ch * BCH
        pltpu.async_copy(h_hbm.at[pl.ds(base, BCH)], h_v, sstg)
        pltpu.async_copy(t_hbm.at[pl.ds(base, BCH)], t_v, sstg)
        pltpu.async_copy(r_hbm.at[pl.ds(base, BCH)], r_v, sstg)
        pltpu.make_async_copy(h_hbm.at[pl.ds(base, BCH)], h_v, sstg).wait()
        pltpu.make_async_copy(t_hbm.at[pl.ds(base, BCH)], t_v, sstg).wait()
        pltpu.make_async_copy(r_hbm.at[pl.ds(base, BCH)], r_v, sstg).wait()
        if ch > 0:
            pbase = base0 + (ch - 1) * BCH
            for j in range(4):
                pltpu.make_async_copy(
                    ex_refs[j], ex_hbm.at[pl.ds(j * E + pbase, BCH)],
                    swb).wait()

        def _edges(i, _):
            sl = pl.ds(i * 16, 16)
            h4 = h_v[sl] * 4
            t4 = t_v[sl] * 4
            rv = r_v[sl]
            rrow = rv >> 5
            rcol = (rv & 31) * 4
            for j in range(4):
                e = (plsc.load_gather(sh_v, [h4 + j])
                     + plsc.load_gather(st_v, [t4 + j]))
                e = jnp.where(e >= 0.0, e, e * 0.01)
                ex = jnp.exp(e - mvec)
                ex_refs[j][sl] = ex
                plsc.addupdate_scatter(ssum_v, [rrow, rcol + j], ex)
            return 0
        lax.fori_loop(0, BCH // 16, _edges, 0)
        for j in range(4):
            pltpu.async_copy(ex_refs[j], ex_hbm.at[pl.ds(j * E + base, BCH)],
                             swb)
    for j in range(4):
        pltpu.make_async_copy(
            ex_refs[j], ex_hbm.at[pl.ds(j * E + base0 + 4 * BCH, BCH)],
            swb).wait()

    pltpu.sync_copy(ssum_v, sacc_sh.at[ridx_v.at[0]], add=True)
    plsc.subcore_barrier()

    @pl.when(sid == 0)
    def _dump():
        pltpu.sync_copy(sacc_sh, ssum_v)
        pltpu.sync_copy(ssum_v, psum_hbm.at[cid])


def _scores(sh_flat, st_flat, m_vec, h_idx, t_idx, rel):
    mesh = plsc.VectorSubcoreMesh(core_axis_name="c", subcore_axis_name="s",
                                  num_cores=NC, num_subcores=NS)
    fn = pl.kernel(
        _scores_body,
        out_type=[
            jax.ShapeDtypeStruct((4 * E,), _f32),
            jax.ShapeDtypeStruct((NC, S2R, 128), _f32),
        ],
        mesh=mesh,
        compiler_params=pltpu.CompilerParams(needs_layout_passes=False),
        scratch_types=[
            pltpu.VMEM((4 * N,), _f32),
            pltpu.VMEM((4 * N,), _f32),
            pltpu.VMEM((16,), _f32),
            pltpu.VMEM((BCH,), _i32),
            pltpu.VMEM((BCH,), _i32),
            pltpu.VMEM((BCH,), _i32),
            pltpu.VMEM((BCH,), _f32),
            pltpu.VMEM((BCH,), _f32),
            pltpu.VMEM((BCH,), _f32),
            pltpu.VMEM((BCH,), _f32),
            pltpu.VMEM((S2R, 128), _f32),
            pltpu.VMEM((2, 128), _f32),
            pltpu.VMEM((1, 32), _i32),
            pltpu.VMEM_SHARED((S2R, 128), _f32),
            pltpu.SemaphoreType.DMA,
            pltpu.SemaphoreType.DMA,
        ],
    )
    return fn(sh_flat, st_flat, m_vec, h_idx, t_idx, rel)


def _agg_body(xhs_hbm, xts_hbm, ex_hbm, h_hbm, t_hbm, r_hbm, psum_hbm,
              outp_hbm,
              inv_v, h_v, t_v, r_v,
              ex0_v, ex1_v, ex2_v, ex3_v,
              bh0_v, bh1_v, bh2_v, bt0_v, bt1_v, bt2_v, acc_v, ridx_v,
              acc_sh, sgh0, sgh1, sgh2, sgt0, sgt1, sgt2, sstg):
    cid = lax.axis_index("c")
    sid = lax.axis_index("s")
    iota = lax.iota(_i32, 16)
    coff = cid * HH

    pltpu.sync_copy(psum_hbm.at[0], bh0_v.at[pl.ds(0, S2R)])

    def _s0(r, _):
        for q in range(8):
            sl = pl.ds(q * 16, 16)
            inv_v[pl.ds(r * 128 + q * 16, 16)] = bh0_v[r, sl]
        return 0
    lax.fori_loop(0, S2R, _s0, 0)
    pltpu.sync_copy(psum_hbm.at[1], bh0_v.at[pl.ds(0, S2R)])

    def _inv(r, _):
        for q in range(8):
            sl = pl.ds(r * 128 + q * 16, 16)
            inv_v[sl] = 1.0 / (inv_v[sl] + bh0_v[r, pl.ds(q * 16, 16)]
                               + 1e-16)
        return 0
    lax.fori_loop(0, S2R, _inv, 0)

    def _zacc(r, _):
        for q in range(8):
            acc_v[r, pl.ds(q * 16, 16)] = jnp.zeros((16,), _f32)
        return 0
    lax.fori_loop(0, RP // 2, _zacc, 0)
    pltpu.sync_copy(acc_v.at[pl.ds(0, 32)], acc_sh.at[pl.ds(sid * 32, 32)])
    for k in range(4):
        for q in range(8):
            ridx_v[k, pl.ds(q * 16, 16)] = k * 128 + q * 16 + iota
    plsc.subcore_barrier()

    bh = (bh0_v, bh1_v, bh2_v)
    bt = (bt0_v, bt1_v, bt2_v)
    sgh = (sgh0, sgh1, sgh2)
    sgt = (sgt0, sgt1, sgt2)
    ex_refs = (ex0_v, ex1_v, ex2_v, ex3_v)
    base0 = sid * EPT

    def _gissue(slot, sub):
        gb = pl.ds(sub * CSUB, CSUB)
        pltpu.async_copy(xhs_hbm.at[h_v.at[gb]], bh[slot], sgh[slot])
        pltpu.async_copy(xts_hbm.at[t_v.at[gb]], bt[slot], sgt[slot])

    def _gwait(slot, sub):
        gb = pl.ds(sub * CSUB, CSUB)
        pltpu.make_async_copy(xhs_hbm.at[h_v.at[gb]], bh[slot], sgh[slot]).wait()
        pltpu.make_async_copy(xts_hbm.at[t_v.at[gb]], bt[slot], sgt[slot]).wait()

    iotav = [iota + v * 16 for v in range(4)]

    def _chunk(ch, _):
        base = base0 + ch * BCH
        pltpu.async_copy(h_hbm.at[pl.ds(base, BCH)], h_v, sstg)
        pltpu.async_copy(t_hbm.at[pl.ds(base, BCH)], t_v, sstg)
        pltpu.async_copy(r_hbm.at[pl.ds(base, BCH)], r_v, sstg)
        for j in range(4):
            pltpu.async_copy(ex_hbm.at[pl.ds(j * E + base, BCH)],
                             ex_refs[j], sstg)
        pltpu.make_async_copy(h_hbm.at[pl.ds(base, BCH)], h_v, sstg).wait()
        pltpu.make_async_copy(t_hbm.at[pl.ds(base, BCH)], t_v, sstg).wait()
        pltpu.make_async_copy(r_hbm.at[pl.ds(base, BCH)], r_v, sstg).wait()
        for j in range(4):
            pltpu.make_async_copy(ex_hbm.at[pl.ds(j * E + base, BCH)],
                                  ex_refs[j], sstg).wait()

        def _wcomb(i, _):
            sl = pl.ds(i * 16, 16)
            r4 = r_v[sl] * 4
            i0 = plsc.load_gather(inv_v, [r4])
            i1 = plsc.load_gather(inv_v, [r4 + 1])
            i2 = plsc.load_gather(inv_v, [r4 + 2])
            i3 = plsc.load_gather(inv_v, [r4 + 3])
            ex0_v[sl] = ex0_v[sl] * i0 + ex1_v[sl] * i1
            ex2_v[sl] = ex2_v[sl] * i2 + ex3_v[sl] * i3
            return 0
        lax.fori_loop(0, BCH // 16, _wcomb, 0)

        _gissue(0, 0)
        _gissue(1, 1)
        _gissue(2, 2)

        def _do_sub(slot, sub):
            _gwait(slot, sub)

            def _edge(e5, _):
                for u in range(5):
                    e = e5 * 5 + u
                    eb = jnp.full((16,), sub * CSUB + e, _i32)
                    relb = plsc.load_gather(r_v, [eb])
                    whb = plsc.load_gather(ex0_v, [eb])
                    wtb = plsc.load_gather(ex2_v, [eb])
                    arow = relb >> 1
                    acolb = (relb & 1) * HH
                    for v in range(4):
                        slv = pl.ds(coff + v * 16, 16)
                        acol = acolb + iotav[v]
                        ph = bh[slot][e, slv] * whb
                        pt = bt[slot][e, slv] * wtb
                        plsc.addupdate_scatter(acc_v, [arow, acol], ph)
                        plsc.addupdate_scatter(acc_v, [arow, acol], pt)
                return 0
            lax.fori_loop(0, CSUB // 5, _edge, 0)

        def _trip(p, _):
            for b in range(3):
                sub = p * 3 + b
                _do_sub(b, sub)
                _gissue(b, sub + 3)
            return 0
        lax.fori_loop(0, (NSUB - 5) // 3, _trip, 0)
        _do_sub(0, NSUB - 5)
        _gissue(0, NSUB - 2)
        _do_sub(1, NSUB - 4)
        _gissue(1, NSUB - 1)
        _do_sub(2, NSUB - 3)
        _do_sub(0, NSUB - 2)
        _do_sub(1, NSUB - 1)
        return 0
    lax.fori_loop(0, EPT // BCH, _chunk, 0)

    plsc.subcore_barrier()
    for k in range(4):
        pltpu.sync_copy(acc_v.at[pl.ds(k * 128, 128)],
                        acc_sh.at[ridx_v.at[k]], add=True)
    plsc.subcore_barrier()
    pltpu.sync_copy(acc_sh.at[pl.ds(sid * 32, 32)], acc_v.at[pl.ds(0, 32)])
    pltpu.sync_copy(acc_v.at[pl.ds(0, 32)], outp_hbm.at[cid, pl.ds(sid * 32, 32)])


def _aggregate(xhs, xts, ex_flat, h_idx, t_idx, rel, psum):
    mesh = plsc.VectorSubcoreMesh(core_axis_name="c", subcore_axis_name="s",
                                  num_cores=NC, num_subcores=NS)
    fn = pl.kernel(
        _agg_body,
        out_type=jax.ShapeDtypeStruct((NC, RP // 2, HID), _f32),
        mesh=mesh,
        compiler_params=pltpu.CompilerParams(needs_layout_passes=False),
        scratch_types=[
            pltpu.VMEM((S2R * 128,), _f32),
            pltpu.VMEM((BCH,), _i32),
            pltpu.VMEM((BCH,), _i32),
            pltpu.VMEM((BCH,), _i32),
            pltpu.VMEM((BCH,), _f32),
            pltpu.VMEM((BCH,), _f32),
            pltpu.VMEM((BCH,), _f32),
            pltpu.VMEM((BCH,), _f32),
            pltpu.VMEM((CSUB, HID), _f32),
            pltpu.VMEM((CSUB, HID), _f32),
            pltpu.VMEM((CSUB, HID), _f32),
            pltpu.VMEM((CSUB, HID), _f32),
            pltpu.VMEM((CSUB, HID), _f32),
            pltpu.VMEM((CSUB, HID), _f32),
            pltpu.VMEM((RP // 2, HID), _f32),
            pltpu.VMEM((4, 128), _i32),
            pltpu.VMEM_SHARED((RP // 2, HID), _f32),
            pltpu.SemaphoreType.DMA,
            pltpu.SemaphoreType.DMA,
            pltpu.SemaphoreType.DMA,
            pltpu.SemaphoreType.DMA,
            pltpu.SemaphoreType.DMA,
            pltpu.SemaphoreType.DMA,
            pltpu.SemaphoreType.DMA,
        ],
    )
    return fn(xhs, xts, ex_flat, h_idx, t_idx, rel, psum)


def _final_body(p_ref, s_ref, o_ref):
    s = s_ref[0, 0]
    o_ref[...] = jnp.concatenate(
        [p_ref[0, :RELS, :], p_ref[1, :RELS, :]], axis=1) * s


def _finalize(outp, scale):
    return pl.pallas_call(
        _final_body,
        out_shape=jax.ShapeDtypeStruct((RELS, HID), _f32),
    )(outp, scale)


def kernel(x_e, edge_index, rel, num_heads, w_h, w_t, a_h, a_t):
    h_idx = edge_index[0]
    t_idx = edge_index[1]
    rel = rel.astype(_i32)
    xhs, xts, sh, st, m = _project(x_e, w_h, w_t, a_h, a_t)
    ex_flat, psum = _scores(sh.reshape(-1), st.reshape(-1), m.reshape(16),
                            h_idx, t_idx, rel)
    outp = _aggregate(xhs, xts, ex_flat, h_idx, t_idx, rel, psum)
    outp = outp.reshape(NC, RP, HH)
    scale = (0.5 / jnp.asarray(num_heads, _f32)).reshape(1, 1)
    return _finalize(outp, scale)
